# baseline (speedup 1.0000x reference)
import jax
import jax.numpy as jnp
from jax import lax
from jax.experimental import pallas as pl
from jax.experimental.pallas import tpu as pltpu
from jax.experimental.pallas import tpu_sc as plsc

N = 10000
E = 320000
F = 16
NW = 32
NSUB = 16
EW = E // NW
CH = 128
CHS = 8
NCH = (EW + CH * CHS - 1) // (CH * CHS)
EP = NCH * CHS * CH
NBUF = 4
NPAD = 10112
RPS = NPAD // NSUB
DUMMY = N


def _mesh():
    return plsc.VectorSubcoreMesh(core_axis_name="c", subcore_axis_name="s")


def _deg_body(col_hbm, deg_out, colv, degv):
    cid = lax.axis_index("c")
    sid = lax.axis_index("s")
    wid = cid * NSUB + sid
    pltpu.sync_copy(col_hbm.at[wid], colv)

    def zero(i, _):
        degv[pl.ds(i * 16, 16)] = jnp.zeros((16,), jnp.float32)
        return 0

    lax.fori_loop(0, NPAD // 16, zero, 0)
    ones = jnp.ones((16,), jnp.float32)

    def upd(i, _):
        idx = colv[pl.ds(i * 16, 16)]
        plsc.addupdate_scatter(degv, [idx], ones)
        return 0

    lax.fori_loop(0, EP // 16, upd, 0)
    pltpu.sync_copy(degv, deg_out.at[wid])


_deg_call = pl.kernel(
    _deg_body,
    out_type=jax.ShapeDtypeStruct((NW, NPAD), jnp.float32),
    mesh=_mesh(),
    scratch_types=[
        pltpu.VMEM((EP,), jnp.int32),
        pltpu.VMEM((NPAD,), jnp.float32),
    ],
    compiler_params=pltpu.CompilerParams(needs_layout_passes=False),
)


def _msg_body(y_hbm, row_hbm, col_hbm, zero_hbm, out_hbm,
              rowv, colv, gbuf0, gbuf1, gbuf2, gbuf3, shacc, semg, sems):
    cid = lax.axis_index("c")
    sid = lax.axis_index("s")
    wid = cid * NSUB + sid
    pltpu.sync_copy(row_hbm.at[wid], rowv)
    pltpu.sync_copy(col_hbm.at[wid], colv)
    pltpu.sync_copy(zero_hbm.at[pl.ds(sid * RPS, RPS)], shacc.at[pl.ds(sid * RPS, RPS)])
    plsc.subcore_barrier()

    bufs = [gbuf0, gbuf1, gbuf2, gbuf3]
    gd = [None] * NCH
    sd = [None] * NCH
    gd[0] = pltpu.async_copy(y_hbm.at[rowv.at[0]], bufs[0], semg)
    for j in range(NCH):
        gd[j].wait()
        if j + 1 < NCH:
            if j + 1 >= NBUF:
                sd[j + 1 - NBUF].wait()
            gd[j + 1] = pltpu.async_copy(
                y_hbm.at[rowv.at[j + 1]], bufs[(j + 1) % NBUF], semg)
        sd[j] = pltpu.async_copy(
            bufs[j % NBUF], shacc.at[colv.at[j]], sems, add=True)
    for j in range(max(0, NCH - NBUF), NCH):
        sd[j].wait()
    plsc.subcore_barrier()
    pltpu.sync_copy(shacc.at[pl.ds(sid * RPS, RPS)],
                    out_hbm.at[cid, pl.ds(sid * RPS, RPS)])


_msg_call = pl.kernel(
    _msg_body,
    out_type=jax.ShapeDtypeStruct((2, NPAD, F), jnp.float32),
    mesh=_mesh(),
    scratch_types=[
        pltpu.VMEM((NCH, CHS * CH), jnp.int32),
        pltpu.VMEM((NCH, CHS * CH), jnp.int32),
        pltpu.VMEM((CHS * CH, F), jnp.float32),
        pltpu.VMEM((CHS * CH, F), jnp.float32),
        pltpu.VMEM((CHS * CH, F), jnp.float32),
        pltpu.VMEM((CHS * CH, F), jnp.float32),
        pltpu.VMEM_SHARED((NPAD, F), jnp.float32),
        pltpu.SemaphoreType.DMA,
        pltpu.SemaphoreType.DMA,
    ],
    compiler_params=pltpu.CompilerParams(use_tc_tiling_on_sc=False),
)


def _tc1_body(x_ref, wnn_ref, bnn_ref, glob_ref, wgn_ref, bgn_ref, degp_ref,
              y1_ref, dis_ref):
    deg = jnp.sum(degp_ref[...], axis=0)[:N] + 1.0
    dis16 = jnp.broadcast_to(lax.rsqrt(deg)[:, None], (N, F))
    gl = glob_ref[...] @ wgn_ref[...] + bgn_ref[...]
    x1 = x_ref[...] @ wnn_ref[...] + bnn_ref[...] + gl
    dis_ref[...] = dis16
    y1_ref[...] = dis16 * x1


def _tc2_body(acc_ref, y1_ref, dis_ref, glob_ref, wgg_ref, bgg_ref, wng_ref,
              bng_ref, wnn2_ref, bnn2_ref, wgn2_ref, bgn2_ref, y2_ref):
    acc = acc_ref[...]
    dis16 = dis_ref[...]
    out1 = dis16 * (acc[0, :N] + acc[1, :N] + y1_ref[...])
    gp = jnp.max(out1, axis=0, keepdims=True)
    glob2 = (glob_ref[...] @ wgg_ref[...] + bgg_ref[...]
             + gp @ wng_ref[...] + bng_ref[...])
    h1 = jnp.maximum(out1, 0.0)
    x2 = h1 @ wnn2_ref[...] + bnn2_ref[...] + glob2 @ wgn2_ref[...] + bgn2_ref[...]
    y2_ref[...] = dis16 * x2


def _tc3_body(acc_ref, y2_ref, dis_ref, o_ref):
    acc = acc_ref[...]
    out2 = dis_ref[...] * (acc[0, :N] + acc[1, :N] + y2_ref[...])
    o_ref[...] = jax.nn.sigmoid(out2)


def _sds(shape):
    return jax.ShapeDtypeStruct(shape, jnp.float32)


def kernel(x, edge_index, glob_init,
           W_nn1, b_nn1, W_gn1, b_gn1, W_gg1, b_gg1, W_ng1, b_ng1,
           W_nn2, b_nn2, W_gn2, b_gn2, W_gg2, b_gg2, W_ng2, b_ng2):
    row = edge_index[0].reshape(NW, EW)
    col = edge_index[1].reshape(NW, EW)
    pad = EP - EW
    rowc = jnp.pad(row, ((0, 0), (0, pad)), constant_values=0).reshape(NW, NCH, CHS * CH)
    colp = jnp.pad(col, ((0, 0), (0, pad)), constant_values=DUMMY)
    colc = colp.reshape(NW, NCH, CHS * CH)
    zeros_hbm = jnp.zeros((NPAD, F), jnp.float32)
    b_nn1r = b_nn1.reshape(1, F)
    b_gn1r = b_gn1.reshape(1, F)
    b_gg1r = b_gg1.reshape(1, F)
    b_ng1r = b_ng1.reshape(1, F)
    b_nn2r = b_nn2.reshape(1, F)
    b_gn2r = b_gn2.reshape(1, F)

    degp = _deg_call(colp)

    y1, dis16 = pl.pallas_call(
        _tc1_body,
        out_shape=(_sds((N, F)), _sds((N, F))),
    )(x, W_nn1, b_nn1r, glob_init, W_gn1, b_gn1r, degp)

    acc1 = _msg_call(y1, rowc, colc, zeros_hbm)

    y2 = pl.pallas_call(
        _tc2_body,
        out_shape=_sds((N, F)),
    )(acc1, y1, dis16, glob_init, W_gg1, b_gg1r, W_ng1, b_ng1r,
      W_nn2, b_nn2r, W_gn2, b_gn2r)

    acc2 = _msg_call(y2, rowc, colc, zeros_hbm)

    out = pl.pallas_call(
        _tc3_body,
        out_shape=_sds((N, F)),
    )(acc2, y2, dis16)
    return out


# speedup vs baseline: 1.4782x; 1.4782x over previous
import jax
import jax.numpy as jnp
from jax import lax
from jax.experimental import pallas as pl
from jax.experimental.pallas import tpu as pltpu
from jax.experimental.pallas import tpu_sc as plsc

N = 10000
E = 320000
F = 16
NW = 32
NSUB = 16
EW = E // NW
CH = 128
CHS = 8
NCH = (EW + CH * CHS - 1) // (CH * CHS)
EP = NCH * CHS * CH
NBUF = 4
NPAD = 10112
RPS = NPAD // NSUB
DUMMY = N


def _mesh():
    return plsc.VectorSubcoreMesh(core_axis_name="c", subcore_axis_name="s")


def _deg_body(col_hbm, deg_out, colv, degv):
    cid = lax.axis_index("c")
    sid = lax.axis_index("s")
    wid = cid * NSUB + sid
    pltpu.sync_copy(col_hbm.at[wid], colv)

    def zero(i, _):
        degv[pl.ds(i * 16, 16)] = jnp.zeros((16,), jnp.float32)
        return 0

    lax.fori_loop(0, NPAD // 16, zero, 0)
    ones = jnp.ones((16,), jnp.float32)

    def upd(i, _):
        idx = colv[pl.ds(i * 16, 16)]
        plsc.addupdate_scatter(degv, [idx], ones)
        return 0

    lax.fori_loop(0, EP // 16, upd, 0)
    pltpu.sync_copy(degv, deg_out.at[wid])


_deg_call = pl.kernel(
    _deg_body,
    out_type=jax.ShapeDtypeStruct((NW, NPAD), jnp.float32),
    mesh=_mesh(),
    scratch_types=[
        pltpu.VMEM((EP,), jnp.int32),
        pltpu.VMEM((NPAD,), jnp.float32),
    ],
    compiler_params=pltpu.CompilerParams(needs_layout_passes=False),
)


def _msg_body(y_hbm, row_hbm, col_hbm, zero_hbm, out_hbm,
              rowv, colv, gbuf0, gbuf1, gbuf2, gbuf3, shy, shacc, semg, sems):
    cid = lax.axis_index("c")
    sid = lax.axis_index("s")
    wid = cid * NSUB + sid
    pltpu.sync_copy(row_hbm.at[wid], rowv)
    pltpu.sync_copy(col_hbm.at[wid], colv)
    pltpu.sync_copy(y_hbm.at[pl.ds(sid * RPS, RPS)], shy.at[pl.ds(sid * RPS, RPS)])
    pltpu.sync_copy(zero_hbm.at[pl.ds(sid * RPS, RPS)], shacc.at[pl.ds(sid * RPS, RPS)])
    plsc.subcore_barrier()

    bufs = [gbuf0, gbuf1, gbuf2, gbuf3]
    gd = [None] * NCH
    sd = [None] * NCH
    gd[0] = pltpu.async_copy(shy.at[rowv.at[0]], bufs[0], semg)
    for j in range(NCH):
        gd[j].wait()
        if j + 1 < NCH:
            if j + 1 >= NBUF:
                sd[j + 1 - NBUF].wait()
            gd[j + 1] = pltpu.async_copy(
                shy.at[rowv.at[j + 1]], bufs[(j + 1) % NBUF], semg)
        sd[j] = pltpu.async_copy(
            bufs[j % NBUF], shacc.at[colv.at[j]], sems, add=True)
    for j in range(max(0, NCH - NBUF), NCH):
        sd[j].wait()
    plsc.subcore_barrier()
    pltpu.sync_copy(shacc.at[pl.ds(sid * RPS, RPS)],
                    out_hbm.at[cid, pl.ds(sid * RPS, RPS)])


_msg_call = pl.kernel(
    _msg_body,
    out_type=jax.ShapeDtypeStruct((2, NPAD, F), jnp.float32),
    mesh=_mesh(),
    scratch_types=[
        pltpu.VMEM((NCH, CHS * CH), jnp.int32),
        pltpu.VMEM((NCH, CHS * CH), jnp.int32),
        pltpu.VMEM((CHS * CH, F), jnp.float32),
        pltpu.VMEM((CHS * CH, F), jnp.float32),
        pltpu.VMEM((CHS * CH, F), jnp.float32),
        pltpu.VMEM((CHS * CH, F), jnp.float32),
        pltpu.VMEM_SHARED((NPAD, F), jnp.float32),
        pltpu.VMEM_SHARED((NPAD, F), jnp.float32),
        pltpu.SemaphoreType.DMA,
        pltpu.SemaphoreType.DMA,
    ],
    compiler_params=pltpu.CompilerParams(use_tc_tiling_on_sc=False),
)


def _tc1_body(x_ref, wnn_ref, bnn_ref, glob_ref, wgn_ref, bgn_ref, degp_ref,
              y1_ref, dis_ref):
    deg = jnp.sum(degp_ref[...], axis=0)[:N] + 1.0
    dis16 = jnp.broadcast_to(lax.rsqrt(deg)[:, None], (N, F))
    gl = glob_ref[...] @ wgn_ref[...] + bgn_ref[...]
    x1 = x_ref[...] @ wnn_ref[...] + bnn_ref[...] + gl
    dis_ref[...] = dis16
    y1_ref[pl.ds(0, N), :] = dis16 * x1
    y1_ref[pl.ds(N, NPAD - N), :] = jnp.zeros((NPAD - N, F), jnp.float32)


def _tc2_body(acc_ref, y1_ref, dis_ref, glob_ref, wgg_ref, bgg_ref, wng_ref,
              bng_ref, wnn2_ref, bnn2_ref, wgn2_ref, bgn2_ref, y2_ref):
    acc = acc_ref[...]
    dis16 = dis_ref[...]
    out1 = dis16 * (acc[0, :N] + acc[1, :N] + y1_ref[pl.ds(0, N), :])
    gp = jnp.max(out1, axis=0, keepdims=True)
    glob2 = (glob_ref[...] @ wgg_ref[...] + bgg_ref[...]
             + gp @ wng_ref[...] + bng_ref[...])
    h1 = jnp.maximum(out1, 0.0)
    x2 = h1 @ wnn2_ref[...] + bnn2_ref[...] + glob2 @ wgn2_ref[...] + bgn2_ref[...]
    y2_ref[pl.ds(0, N), :] = dis16 * x2
    y2_ref[pl.ds(N, NPAD - N), :] = jnp.zeros((NPAD - N, F), jnp.float32)


def _tc3_body(acc_ref, y2_ref, dis_ref, o_ref):
    acc = acc_ref[...]
    out2 = dis_ref[...] * (acc[0, :N] + acc[1, :N] + y2_ref[pl.ds(0, N), :])
    o_ref[...] = jax.nn.sigmoid(out2)


def _sds(shape):
    return jax.ShapeDtypeStruct(shape, jnp.float32)


def kernel(x, edge_index, glob_init,
           W_nn1, b_nn1, W_gn1, b_gn1, W_gg1, b_gg1, W_ng1, b_ng1,
           W_nn2, b_nn2, W_gn2, b_gn2, W_gg2, b_gg2, W_ng2, b_ng2):
    row = edge_index[0].reshape(NW, EW)
    col = edge_index[1].reshape(NW, EW)
    pad = EP - EW
    rowc = jnp.pad(row, ((0, 0), (0, pad)), constant_values=0).reshape(NW, NCH, CHS * CH)
    colp = jnp.pad(col, ((0, 0), (0, pad)), constant_values=DUMMY)
    colc = colp.reshape(NW, NCH, CHS * CH)
    zeros_hbm = jnp.zeros((NPAD, F), jnp.float32)
    b_nn1r = b_nn1.reshape(1, F)
    b_gn1r = b_gn1.reshape(1, F)
    b_gg1r = b_gg1.reshape(1, F)
    b_ng1r = b_ng1.reshape(1, F)
    b_nn2r = b_nn2.reshape(1, F)
    b_gn2r = b_gn2.reshape(1, F)

    degp = _deg_call(colp)

    y1, dis16 = pl.pallas_call(
        _tc1_body,
        out_shape=(_sds((NPAD, F)), _sds((N, F))),
    )(x, W_nn1, b_nn1r, glob_init, W_gn1, b_gn1r, degp)

    acc1 = _msg_call(y1, rowc, colc, zeros_hbm)

    y2 = pl.pallas_call(
        _tc2_body,
        out_shape=_sds((NPAD, F)),
    )(acc1, y1, dis16, glob_init, W_gg1, b_gg1r, W_ng1, b_ng1r,
      W_nn2, b_nn2r, W_gn2, b_gn2r)

    acc2 = _msg_call(y2, rowc, colc, zeros_hbm)

    out = pl.pallas_call(
        _tc3_body,
        out_shape=_sds((N, F)),
    )(acc2, y2, dis16)
    return out


# speedup vs baseline: 1.7560x; 1.1879x over previous
import jax
import jax.numpy as jnp
from jax import lax
from jax.experimental import pallas as pl
from jax.experimental.pallas import tpu as pltpu
from jax.experimental.pallas import tpu_sc as plsc

N = 10000
E = 320000
F = 16
NW = 32
NSUB = 16
EW = E // NW
CHK = 1000
NCH = EW // CHK
NBUF = 4
NPAD = 10112
RPS = NPAD // NSUB


def _mesh():
    return plsc.VectorSubcoreMesh(core_axis_name="c", subcore_axis_name="s")


def _deg_body(edge_hbm, deg_out, colv, degv):
    cid = lax.axis_index("c")
    sid = lax.axis_index("s")
    wid = cid * NSUB + sid
    pltpu.sync_copy(edge_hbm.at[pl.ds(E + wid * EW, EW)], colv)

    def zero(i, _):
        degv[pl.ds(i * 16, 16)] = jnp.zeros((16,), jnp.float32)
        return 0

    lax.fori_loop(0, NPAD // 16, zero, 0)
    ones = jnp.ones((16,), jnp.float32)

    def upd(i, _):
        idx = colv[pl.ds(i * 16, 16)]
        plsc.addupdate_scatter(degv, [idx], ones)
        return 0

    lax.fori_loop(0, EW // 16, upd, 0)
    pltpu.sync_copy(degv, deg_out.at[wid])


_deg_call = pl.kernel(
    _deg_body,
    out_type=jax.ShapeDtypeStruct((NW, NPAD), jnp.float32),
    mesh=_mesh(),
    scratch_types=[
        pltpu.VMEM((EW,), jnp.int32),
        pltpu.VMEM((NPAD,), jnp.float32),
    ],
    compiler_params=pltpu.CompilerParams(needs_layout_passes=False),
)


def _msg_body(y_hbm, edge_hbm, zero_hbm, out_hbm,
              rowv, colv, gbuf0, gbuf1, gbuf2, gbuf3, shy, shacc,
              semi, semg, sems):
    cid = lax.axis_index("c")
    sid = lax.axis_index("s")
    wid = cid * NSUB + sid
    idxd = []
    for j in range(NCH):
        idxd.append(pltpu.async_copy(
            edge_hbm.at[pl.ds(wid * EW + j * CHK, CHK)], rowv.at[j], semi))
        idxd.append(pltpu.async_copy(
            edge_hbm.at[pl.ds(E + wid * EW + j * CHK, CHK)], colv.at[j], semi))
    pltpu.sync_copy(y_hbm.at[pl.ds(sid * RPS, RPS)], shy.at[pl.ds(sid * RPS, RPS)])
    pltpu.sync_copy(zero_hbm.at[pl.ds(sid * RPS, RPS)], shacc.at[pl.ds(sid * RPS, RPS)])
    for d in idxd:
        d.wait()
    plsc.subcore_barrier()

    bufs = [gbuf0, gbuf1, gbuf2, gbuf3]
    gd = [None] * NCH
    sd = [None] * NCH
    gd[0] = pltpu.async_copy(shy.at[rowv.at[0]], bufs[0], semg)
    for j in range(NCH):
        gd[j].wait()
        if j + 1 < NCH:
            if j + 1 >= NBUF:
                sd[j + 1 - NBUF].wait()
            gd[j + 1] = pltpu.async_copy(
                shy.at[rowv.at[j + 1]], bufs[(j + 1) % NBUF], semg)
        sd[j] = pltpu.async_copy(
            bufs[j % NBUF], shacc.at[colv.at[j]], sems, add=True)
    for j in range(max(0, NCH - NBUF), NCH):
        sd[j].wait()
    plsc.subcore_barrier()
    pltpu.sync_copy(shacc.at[pl.ds(sid * RPS, RPS)],
                    out_hbm.at[cid, pl.ds(sid * RPS, RPS)])


_msg_call = pl.kernel(
    _msg_body,
    out_type=jax.ShapeDtypeStruct((2, NPAD, F), jnp.float32),
    mesh=_mesh(),
    scratch_types=[
        pltpu.VMEM((NCH, CHK), jnp.int32),
        pltpu.VMEM((NCH, CHK), jnp.int32),
        pltpu.VMEM((CHK, F), jnp.float32),
        pltpu.VMEM((CHK, F), jnp.float32),
        pltpu.VMEM((CHK, F), jnp.float32),
        pltpu.VMEM((CHK, F), jnp.float32),
        pltpu.VMEM_SHARED((NPAD, F), jnp.float32),
        pltpu.VMEM_SHARED((NPAD, F), jnp.float32),
        pltpu.SemaphoreType.DMA,
        pltpu.SemaphoreType.DMA,
        pltpu.SemaphoreType.DMA,
    ],
    compiler_params=pltpu.CompilerParams(use_tc_tiling_on_sc=False),
)


def _tc1_body(x_ref, wnn_ref, bnn_ref, glob_ref, wgn_ref, bgn_ref, degp_ref,
              y1_ref, dis_ref):
    deg = jnp.sum(degp_ref[...], axis=0)[:N] + 1.0
    dis16 = jnp.broadcast_to(lax.rsqrt(deg)[:, None], (N, F))
    gl = glob_ref[...] @ wgn_ref[...] + bgn_ref[...]
    x1 = x_ref[...] @ wnn_ref[...] + bnn_ref[...] + gl
    dis_ref[...] = dis16
    y1_ref[pl.ds(0, N), :] = dis16 * x1
    y1_ref[pl.ds(N, NPAD - N), :] = jnp.zeros((NPAD - N, F), jnp.float32)


def _tc2_body(acc_ref, y1_ref, dis_ref, glob_ref, wgg_ref, bgg_ref, wng_ref,
              bng_ref, wnn2_ref, bnn2_ref, wgn2_ref, bgn2_ref, y2_ref):
    acc = acc_ref[...]
    dis16 = dis_ref[...]
    out1 = dis16 * (acc[0, :N] + acc[1, :N] + y1_ref[pl.ds(0, N), :])
    gp = jnp.max(out1, axis=0, keepdims=True)
    glob2 = (glob_ref[...] @ wgg_ref[...] + bgg_ref[...]
             + gp @ wng_ref[...] + bng_ref[...])
    h1 = jnp.maximum(out1, 0.0)
    x2 = h1 @ wnn2_ref[...] + bnn2_ref[...] + glob2 @ wgn2_ref[...] + bgn2_ref[...]
    y2_ref[pl.ds(0, N), :] = dis16 * x2
    y2_ref[pl.ds(N, NPAD - N), :] = jnp.zeros((NPAD - N, F), jnp.float32)


def _tc3_body(acc_ref, y2_ref, dis_ref, o_ref):
    acc = acc_ref[...]
    out2 = dis_ref[...] * (acc[0, :N] + acc[1, :N] + y2_ref[pl.ds(0, N), :])
    o_ref[...] = jax.nn.sigmoid(out2)


def _sds(shape):
    return jax.ShapeDtypeStruct(shape, jnp.float32)


def kernel(x, edge_index, glob_init,
           W_nn1, b_nn1, W_gn1, b_gn1, W_gg1, b_gg1, W_ng1, b_ng1,
           W_nn2, b_nn2, W_gn2, b_gn2, W_gg2, b_gg2, W_ng2, b_ng2):
    edge_flat = edge_index.reshape(2 * E)
    zeros_hbm = jnp.zeros((NPAD, F), jnp.float32)
    b_nn1r = b_nn1.reshape(1, F)
    b_gn1r = b_gn1.reshape(1, F)
    b_gg1r = b_gg1.reshape(1, F)
    b_ng1r = b_ng1.reshape(1, F)
    b_nn2r = b_nn2.reshape(1, F)
    b_gn2r = b_gn2.reshape(1, F)

    degp = _deg_call(edge_flat)

    y1, dis16 = pl.pallas_call(
        _tc1_body,
        out_shape=(_sds((NPAD, F)), _sds((N, F))),
    )(x, W_nn1, b_nn1r, glob_init, W_gn1, b_gn1r, degp)

    acc1 = _msg_call(y1, edge_flat, zeros_hbm)

    y2 = pl.pallas_call(
        _tc2_body,
        out_shape=_sds((NPAD, F)),
    )(acc1, y1, dis16, glob_init, W_gg1, b_gg1r, W_ng1, b_ng1r,
      W_nn2, b_nn2r, W_gn2, b_gn2r)

    acc2 = _msg_call(y2, edge_flat, zeros_hbm)

    out = pl.pallas_call(
        _tc3_body,
        out_shape=_sds((N, F)),
    )(acc2, y2, dis16)
    return out
